# baseline (speedup 1.0000x reference)
import functools

import jax
import jax.numpy as jnp
import numpy as np
from jax import lax
from jax.experimental import pallas as pl
from jax.experimental.pallas import tpu as pltpu
from jax.experimental.pallas import tpu_sc as plsc

_MARGIN = 6.0
_EPSILON = 2.0
_DIM = 128
_EMB_RANGE = (_MARGIN + _EPSILON) / _DIM
_BATCH = 16384
_ENT_D = 2 * _DIM
_NREL = 1000

_NC = 2
_NS = 16
_NW = _NC * _NS
_LANES = 16

_FCOMP = 12288
_FMAT = _BATCH - _FCOMP
_U = 32
_PC = _FCOMP // _NW
_NU = _PC // _U
_MU = 64
_PM = _FMAT // _NW
_NM = _PM // _MU

_SIN_C = (9.99999600e-01, -1.66665526e-01, 8.33240285e-03, -1.98086298e-04,
          2.69971060e-06, -2.03620814e-08)
_COS_C = (9.99999989e-01, -4.99999891e-01, 4.16664892e-02, -1.38878034e-03,
          2.47698803e-05, -2.70789985e-07, 1.72449738e-09)


def _poly_sin(x, t):
    acc = jnp.float32(_SIN_C[-1])
    for c in _SIN_C[-2::-1]:
        acc = acc * t + jnp.float32(c)
    return x * acc


def _poly_cos(t):
    acc = jnp.float32(_COS_C[-1])
    for c in _COS_C[-2::-1]:
        acc = acc * t + jnp.float32(c)
    return acc


def _trig_table_kernel(rel_ref, o_ref):
    phase = rel_ref[...] * np.float32(np.pi / _EMB_RANGE)
    t2 = phase * phase
    o_ref[:, :_DIM] = _poly_cos(t2)
    o_ref[:, _DIM:] = _poly_sin(phase, t2)


def _trig_table(relation_embedding):
    return pl.pallas_call(
        _trig_table_kernel,
        out_shape=jax.ShapeDtypeStruct((_NREL, _ENT_D), jnp.float32),
    )(relation_embedding)


def _sc_score_kernel(ent_hbm, trig_hbm, ih_hbm, it_hbm, ir_hbm,
                     out_part, out_ht, out_rel,
                     idx_v, hb, tb, gb, part, mhb, mtb, mrb,
                     gsem, wsem, mgsem, mwsem):
    wid = lax.axis_index("s") * _NC + lax.axis_index("c")
    cbase = wid * _PC
    mbase = wid * _PM
    ih = [pltpu.async_copy(ih_hbm.at[pl.ds(cbase, _PC)],
                           idx_v.at[pl.ds(0, _PC)], gsem),
          pltpu.async_copy(it_hbm.at[pl.ds(cbase, _PC)],
                           idx_v.at[pl.ds(_PC, _PC)], gsem),
          pltpu.async_copy(ir_hbm.at[pl.ds(cbase, _PC)],
                           idx_v.at[pl.ds(2 * _PC, _PC)], gsem),
          pltpu.async_copy(ih_hbm.at[pl.ds(_FCOMP + mbase, _PM)],
                           idx_v.at[pl.ds(3 * _PC, _PM)], gsem),
          pltpu.async_copy(it_hbm.at[pl.ds(_FCOMP + mbase, _PM)],
                           idx_v.at[pl.ds(3 * _PC + _PM, _PM)], gsem),
          pltpu.async_copy(ir_hbm.at[pl.ds(_FCOMP + mbase, _PM)],
                           idx_v.at[pl.ds(3 * _PC + 2 * _PM, _PM)], gsem)]
    for h in ih:
        h.wait()

    def start_unit(u):
        b = u % 2
        off = u * _U
        return (
            pltpu.async_copy(ent_hbm.at[idx_v.at[pl.ds(off, _U)]],
                             hb[b], gsem),
            pltpu.async_copy(ent_hbm.at[idx_v.at[pl.ds(_PC + off, _U)]],
                             tb[b], gsem),
            pltpu.async_copy(trig_hbm.at[idx_v.at[pl.ds(2 * _PC + off, _U)]],
                             gb[b], gsem),
        )

    def start_mat_gather(m):
        off = 3 * _PC + m * _MU
        return (
            pltpu.async_copy(ent_hbm.at[idx_v.at[pl.ds(off, _MU)]],
                             mhb, mgsem),
            pltpu.async_copy(ent_hbm.at[idx_v.at[pl.ds(_PM + off, _MU)]],
                             mtb, mgsem),
            pltpu.async_copy(trig_hbm.at[idx_v.at[pl.ds(2 * _PM + off,
                                                        _MU)]],
                             mrb, mgsem),
        )

    def start_mat_write(m):
        off = mbase + m * _MU
        return (
            pltpu.async_copy(mhb, out_ht.at[pl.ds(off, _MU)], mwsem),
            pltpu.async_copy(mtb, out_ht.at[pl.ds(_FMAT + off, _MU)], mwsem),
            pltpu.async_copy(mrb, out_rel.at[pl.ds(off, _MU)], mwsem),
        )

    mg = [None] * _NM
    mw = [None] * _NM
    mg[0] = start_mat_gather(0)

    gh = [None] * _NU
    wh = [None] * _NU
    gh[0] = start_unit(0)
    for u in range(_NU):
        if u + 1 < _NU:
            gh[u + 1] = start_unit(u + 1)
        for h in gh[u]:
            h.wait()
        if u >= 2:
            wh[u - 2].wait()
        b = u % 2
        hbuf, tbuf, gbuf, pbuf = hb[b], tb[b], gb[b], part[b]

        @pl.loop(0, _U)
        def _(r):
            acc = jnp.zeros((_LANES,), jnp.float32)
            for c in range(_DIM // _LANES):
                lo = c * _LANES
                hi = _DIM + lo
                re_h = hbuf[r, pl.ds(lo, _LANES)]
                im_h = hbuf[r, pl.ds(hi, _LANES)]
                re_t = tbuf[r, pl.ds(lo, _LANES)]
                im_t = tbuf[r, pl.ds(hi, _LANES)]
                cr = gbuf[r, pl.ds(lo, _LANES)]
                sr = gbuf[r, pl.ds(hi, _LANES)]
                d_re = re_h * cr - im_h * sr - re_t
                d_im = re_h * sr + im_h * cr - im_t
                s = d_re * d_re + d_im * d_im
                acc = acc + s * s
            pbuf[pl.ds(r * _LANES, _LANES)] = acc

        wh[u] = pltpu.async_copy(
            pbuf,
            out_part.at[pl.ds((cbase + u * _U) * _LANES, _U * _LANES)],
            wsem)
        if u == 5:
            for h in mg[0]:
                h.wait()
            mw[0] = start_mat_write(0)
        if u == 7:
            for h in mw[0]:
                h.wait()
            mg[1] = start_mat_gather(1)
        if u == 10:
            for h in mg[1]:
                h.wait()
            mw[1] = start_mat_write(1)
    for h in mw[1]:
        h.wait()
    if _NU >= 2:
        wh[_NU - 2].wait()
    wh[_NU - 1].wait()


def _sc_score(entity_embedding, trig, heads, tails, relations):
    mesh = plsc.VectorSubcoreMesh(core_axis_name="c", subcore_axis_name="s")
    run = pl.kernel(
        _sc_score_kernel,
        out_type=(
            jax.ShapeDtypeStruct((_FCOMP * _LANES,), jnp.float32),
            jax.ShapeDtypeStruct((2 * _FMAT, _ENT_D), jnp.float32),
            jax.ShapeDtypeStruct((_FMAT, _ENT_D), jnp.float32),
        ),
        mesh=mesh,
        scratch_types=[
            pltpu.VMEM((3 * _PC + 3 * _PM,), jnp.int32),
            tuple(pltpu.VMEM((_U, _ENT_D), jnp.float32) for _ in range(2)),
            tuple(pltpu.VMEM((_U, _ENT_D), jnp.float32) for _ in range(2)),
            tuple(pltpu.VMEM((_U, _ENT_D), jnp.float32) for _ in range(2)),
            tuple(pltpu.VMEM((_U * _LANES,), jnp.float32) for _ in range(2)),
            pltpu.VMEM((_MU, _ENT_D), jnp.float32),
            pltpu.VMEM((_MU, _ENT_D), jnp.float32),
            pltpu.VMEM((_MU, _ENT_D), jnp.float32),
            pltpu.SemaphoreType.DMA,
            pltpu.SemaphoreType.DMA,
            pltpu.SemaphoreType.DMA,
            pltpu.SemaphoreType.DMA,
        ],
    )
    return run(entity_embedding, trig, heads, tails, relations)


_BB = 2048


def _tc_score_kernel(h_ref, t_ref, g_ref, o_ref):
    re_h = h_ref[:, :_DIM]
    im_h = h_ref[:, _DIM:]
    re_t = t_ref[:, :_DIM]
    im_t = t_ref[:, _DIM:]
    cr = g_ref[:, :_DIM]
    sr = g_ref[:, _DIM:]
    d_re = re_h * cr - im_h * sr - re_t
    d_im = re_h * sr + im_h * cr - im_t
    score = d_re * d_re + d_im * d_im
    acc = jnp.sum(score * score, axis=1)
    o_ref[...] = _MARGIN - jnp.sqrt(acc)


def _tc_score(ht, trig_g):
    nblk = _FMAT // _BB
    return pl.pallas_call(
        _tc_score_kernel,
        grid=(nblk,),
        in_specs=[
            pl.BlockSpec((_BB, _ENT_D), lambda i: (i, 0)),
            pl.BlockSpec((_BB, _ENT_D), lambda i: (i + nblk, 0)),
            pl.BlockSpec((_BB, _ENT_D), lambda i: (i, 0)),
        ],
        out_specs=pl.BlockSpec((_BB,), lambda i: (i,)),
        out_shape=jax.ShapeDtypeStruct((_FMAT,), jnp.float32),
        compiler_params=pltpu.CompilerParams(
            dimension_semantics=("parallel",)),
    )(ht, ht, trig_g)


_GRP = 128 // _LANES


def _finish_kernel(p_ref, o_ref):
    x = p_ref[...]
    k = lax.broadcasted_iota(jnp.int32, (128, _GRP), 0) // _LANES
    g = lax.broadcasted_iota(jnp.int32, (128, _GRP), 1)
    m = (k == g).astype(jnp.bfloat16)
    s = jax.lax.dot_general(x.astype(jnp.bfloat16), m,
                            (((1,), (0,)), ((), ())),
                            preferred_element_type=jnp.float32)
    o_ref[...] = _MARGIN - jnp.sqrt(s)


def _finish(part):
    p2 = part.reshape(_FCOMP // _GRP, 128)
    out = pl.pallas_call(
        _finish_kernel,
        out_shape=jax.ShapeDtypeStruct((_FCOMP // _GRP, _GRP), jnp.float32),
    )(p2)
    return out.reshape(_FCOMP)


@jax.jit
def kernel(heads, relations, tails, entity_embedding, relation_embedding):
    heads = heads.astype(jnp.int32)
    tails = tails.astype(jnp.int32)
    relations = relations.astype(jnp.int32)
    trig = _trig_table(relation_embedding)
    part, ht_mat, trig_mat = _sc_score(entity_embedding, trig, heads, tails,
                                       relations)
    out_sc = _finish(part)
    out_tc = _tc_score(ht_mat, trig_mat)
    return jnp.concatenate([out_sc, out_tc])


# speedup vs baseline: 1.2132x; 1.2132x over previous
import functools

import jax
import jax.numpy as jnp
import numpy as np
from jax import lax
from jax.experimental import pallas as pl
from jax.experimental.pallas import tpu as pltpu
from jax.experimental.pallas import tpu_sc as plsc

_MARGIN = 6.0
_EPSILON = 2.0
_DIM = 128
_EMB_RANGE = (_MARGIN + _EPSILON) / _DIM
_BATCH = 16384
_ENT_D = 2 * _DIM
_NREL = 1000

_NC = 2
_NS = 16
_NW = _NC * _NS
_LANES = 16

_U = 64
_PER_W = _BATCH // _NW
_NU = _PER_W // _U

_SIN_C = (9.99999600e-01, -1.66665526e-01, 8.33240285e-03, -1.98086298e-04,
          2.69971060e-06, -2.03620814e-08)
_COS_C = (9.99999989e-01, -4.99999891e-01, 4.16664892e-02, -1.38878034e-03,
          2.47698803e-05, -2.70789985e-07, 1.72449738e-09)


def _poly_sin(x, t):
    acc = jnp.float32(_SIN_C[-1])
    for c in _SIN_C[-2::-1]:
        acc = acc * t + jnp.float32(c)
    return x * acc


def _poly_cos(t):
    acc = jnp.float32(_COS_C[-1])
    for c in _COS_C[-2::-1]:
        acc = acc * t + jnp.float32(c)
    return acc


def _trig_table_kernel(rel_ref, o_ref):
    phase = rel_ref[...] * np.float32(np.pi / _EMB_RANGE)
    t2 = phase * phase
    o_ref[:, :_DIM] = _poly_cos(t2)
    o_ref[:, _DIM:] = _poly_sin(phase, t2)


def _trig_table(relation_embedding):
    return pl.pallas_call(
        _trig_table_kernel,
        out_shape=jax.ShapeDtypeStruct((_NREL, _ENT_D), jnp.float32),
    )(relation_embedding)


def _sc_score_kernel(ent_hbm, trig_hbm, ih_hbm, it_hbm, ir_hbm, out_part,
                     idx_v, hb, tb, gb, part, gsem, wsem):
    wid = lax.axis_index("s") * _NC + lax.axis_index("c")
    base = wid * _PER_W
    ih = [pltpu.async_copy(ih_hbm.at[pl.ds(base, _PER_W)],
                           idx_v.at[pl.ds(0, _PER_W)], gsem),
          pltpu.async_copy(it_hbm.at[pl.ds(base, _PER_W)],
                           idx_v.at[pl.ds(_PER_W, _PER_W)], gsem),
          pltpu.async_copy(ir_hbm.at[pl.ds(base, _PER_W)],
                           idx_v.at[pl.ds(2 * _PER_W, _PER_W)], gsem)]
    for h in ih:
        h.wait()

    def start_unit(u):
        b = u % 2
        off = u * _U
        return (
            pltpu.async_copy(ent_hbm.at[idx_v.at[pl.ds(off, _U)]],
                             hb[b], gsem),
            pltpu.async_copy(ent_hbm.at[idx_v.at[pl.ds(_PER_W + off, _U)]],
                             tb[b], gsem),
            pltpu.async_copy(trig_hbm.at[idx_v.at[pl.ds(2 * _PER_W + off,
                                                        _U)]],
                             gb[b], gsem),
        )

    gh = [None] * _NU
    wh = [None] * _NU
    gh[0] = start_unit(0)
    for u in range(_NU):
        if u + 1 < _NU:
            gh[u + 1] = start_unit(u + 1)
        for h in gh[u]:
            h.wait()
        if u >= 2:
            wh[u - 2].wait()
        b = u % 2
        hbuf, tbuf, gbuf, pbuf = hb[b], tb[b], gb[b], part[b]

        @pl.loop(0, _U)
        def _(r):
            acc = jnp.zeros((_LANES,), jnp.float32)
            for c in range(_DIM // _LANES):
                lo = c * _LANES
                hi = _DIM + lo
                re_h = hbuf[r, pl.ds(lo, _LANES)]
                im_h = hbuf[r, pl.ds(hi, _LANES)]
                re_t = tbuf[r, pl.ds(lo, _LANES)]
                im_t = tbuf[r, pl.ds(hi, _LANES)]
                cr = gbuf[r, pl.ds(lo, _LANES)]
                sr = gbuf[r, pl.ds(hi, _LANES)]
                d_re = re_h * cr - im_h * sr - re_t
                d_im = re_h * sr + im_h * cr - im_t
                s = d_re * d_re + d_im * d_im
                acc = acc + s * s
            pbuf[pl.ds(r * _LANES, _LANES)] = acc

        wh[u] = pltpu.async_copy(
            pbuf, out_part.at[pl.ds((base + u * _U) * _LANES, _U * _LANES)],
            wsem)
    if _NU >= 2:
        wh[_NU - 2].wait()
    wh[_NU - 1].wait()


def _sc_score(entity_embedding, trig, heads, tails, relations):
    mesh = plsc.VectorSubcoreMesh(core_axis_name="c", subcore_axis_name="s")
    run = pl.kernel(
        _sc_score_kernel,
        out_type=jax.ShapeDtypeStruct((_BATCH * _LANES,), jnp.float32),
        mesh=mesh,
        scratch_types=[
            pltpu.VMEM((3 * _PER_W,), jnp.int32),
            tuple(pltpu.VMEM((_U, _ENT_D), jnp.float32) for _ in range(2)),
            tuple(pltpu.VMEM((_U, _ENT_D), jnp.float32) for _ in range(2)),
            tuple(pltpu.VMEM((_U, _ENT_D), jnp.float32) for _ in range(2)),
            tuple(pltpu.VMEM((_U * _LANES,), jnp.float32) for _ in range(2)),
            pltpu.SemaphoreType.DMA,
            pltpu.SemaphoreType.DMA,
        ],
    )
    return run(entity_embedding, trig, heads, tails, relations)


_GRP = 128 // _LANES


def _finish_kernel(p_ref, o_ref):
    x = p_ref[...]
    k = lax.broadcasted_iota(jnp.int32, (128, _GRP), 0) // _LANES
    g = lax.broadcasted_iota(jnp.int32, (128, _GRP), 1)
    m = (k == g).astype(jnp.bfloat16)
    s = jax.lax.dot_general(x.astype(jnp.bfloat16), m, (((1,), (0,)), ((), ())),
                            preferred_element_type=jnp.float32)
    o_ref[...] = _MARGIN - jnp.sqrt(s)


def _finish(part):
    p2 = part.reshape(_BATCH // _GRP, 128)
    out = pl.pallas_call(
        _finish_kernel,
        out_shape=jax.ShapeDtypeStruct((_BATCH // _GRP, _GRP), jnp.float32),
    )(p2)
    return out.reshape(_BATCH)


@jax.jit
def kernel(heads, relations, tails, entity_embedding, relation_embedding):
    heads = heads.astype(jnp.int32)
    tails = tails.astype(jnp.int32)
    relations = relations.astype(jnp.int32)
    trig = _trig_table(relation_embedding)
    part = _sc_score(entity_embedding, trig, heads, tails, relations)
    return _finish(part)


# speedup vs baseline: 1.2498x; 1.0302x over previous
import functools

import jax
import jax.numpy as jnp
import numpy as np
from jax import lax
from jax.experimental import pallas as pl
from jax.experimental.pallas import tpu as pltpu
from jax.experimental.pallas import tpu_sc as plsc

_MARGIN = 6.0
_EPSILON = 2.0
_DIM = 128
_EMB_RANGE = (_MARGIN + _EPSILON) / _DIM
_BATCH = 16384
_ENT_D = 2 * _DIM
_NREL = 1000

_NC = 2
_NS = 16
_NW = _NC * _NS
_LANES = 16

_U = 64
_PER_W = _BATCH // _NW
_NU = _PER_W // _U

_SIN_C = (9.99999600e-01, -1.66665526e-01, 8.33240285e-03, -1.98086298e-04,
          2.69971060e-06, -2.03620814e-08)
_COS_C = (9.99999989e-01, -4.99999891e-01, 4.16664892e-02, -1.38878034e-03,
          2.47698803e-05, -2.70789985e-07, 1.72449738e-09)


def _poly_sin(x, t):
    acc = jnp.float32(_SIN_C[-1])
    for c in _SIN_C[-2::-1]:
        acc = acc * t + jnp.float32(c)
    return x * acc


def _poly_cos(t):
    acc = jnp.float32(_COS_C[-1])
    for c in _COS_C[-2::-1]:
        acc = acc * t + jnp.float32(c)
    return acc


def _trig_table_kernel(rel_ref, o_ref):
    phase = rel_ref[...] * np.float32(np.pi / _EMB_RANGE)
    t2 = phase * phase
    o_ref[:, :_DIM] = _poly_cos(t2)
    o_ref[:, _DIM:] = _poly_sin(phase, t2)


def _trig_table(relation_embedding):
    return pl.pallas_call(
        _trig_table_kernel,
        out_shape=jax.ShapeDtypeStruct((_NREL, _ENT_D), jnp.float32),
    )(relation_embedding)


def _sc_score_kernel(ent_hbm, trig_hbm, ih_hbm, it_hbm, ir_hbm, out_part,
                     idx_v, hb, tb, gb, part, gsem, wsem):
    wid = lax.axis_index("s") * _NC + lax.axis_index("c")
    base = wid * _PER_W
    ih = [pltpu.async_copy(ih_hbm.at[pl.ds(base, _PER_W)],
                           idx_v.at[pl.ds(0, _PER_W)], gsem),
          pltpu.async_copy(it_hbm.at[pl.ds(base, _PER_W)],
                           idx_v.at[pl.ds(_PER_W, _PER_W)], gsem),
          pltpu.async_copy(ir_hbm.at[pl.ds(base, _PER_W)],
                           idx_v.at[pl.ds(2 * _PER_W, _PER_W)], gsem)]
    for h in ih:
        h.wait()

    def start_unit(u):
        b = u % 2
        off = u * _U
        return (
            pltpu.async_copy(ent_hbm.at[idx_v.at[pl.ds(off, _U)]],
                             hb[b], gsem),
            pltpu.async_copy(ent_hbm.at[idx_v.at[pl.ds(_PER_W + off, _U)]],
                             tb[b], gsem),
            pltpu.async_copy(trig_hbm.at[idx_v.at[pl.ds(2 * _PER_W + off,
                                                        _U)]],
                             gb[b], gsem),
        )

    gh = [None] * _NU
    wh = [None] * _NU
    gh[0] = start_unit(0)
    for u in range(_NU):
        if u + 1 < _NU:
            gh[u + 1] = start_unit(u + 1)
        for h in gh[u]:
            h.wait()
        if u >= 2:
            wh[u - 2].wait()
        b = u % 2
        hbuf, tbuf, gbuf, pbuf = hb[b], tb[b], gb[b], part[b]

        @pl.loop(0, _U // 2)
        def _(r2):
            r = r2 * 2
            for dr in range(2):
                acc = jnp.zeros((_LANES,), jnp.float32)
                for c in range(_DIM // _LANES):
                    lo = c * _LANES
                    hi = _DIM + lo
                    re_h = hbuf[r + dr, pl.ds(lo, _LANES)]
                    im_h = hbuf[r + dr, pl.ds(hi, _LANES)]
                    re_t = tbuf[r + dr, pl.ds(lo, _LANES)]
                    im_t = tbuf[r + dr, pl.ds(hi, _LANES)]
                    cr = gbuf[r + dr, pl.ds(lo, _LANES)]
                    sr = gbuf[r + dr, pl.ds(hi, _LANES)]
                    d_re = re_h * cr - im_h * sr - re_t
                    d_im = re_h * sr + im_h * cr - im_t
                    s = d_re * d_re + d_im * d_im
                    acc = acc + s * s
                pbuf[pl.ds((r + dr) * _LANES, _LANES)] = acc

        wh[u] = pltpu.async_copy(
            pbuf, out_part.at[pl.ds((base + u * _U) * _LANES, _U * _LANES)],
            wsem)
    if _NU >= 2:
        wh[_NU - 2].wait()
    wh[_NU - 1].wait()


def _sc_score(entity_embedding, trig, heads, tails, relations):
    mesh = plsc.VectorSubcoreMesh(core_axis_name="c", subcore_axis_name="s")
    run = pl.kernel(
        _sc_score_kernel,
        out_type=jax.ShapeDtypeStruct((_BATCH * _LANES,), jnp.float32),
        mesh=mesh,
        scratch_types=[
            pltpu.VMEM((3 * _PER_W,), jnp.int32),
            tuple(pltpu.VMEM((_U, _ENT_D), jnp.float32) for _ in range(2)),
            tuple(pltpu.VMEM((_U, _ENT_D), jnp.float32) for _ in range(2)),
            tuple(pltpu.VMEM((_U, _ENT_D), jnp.float32) for _ in range(2)),
            tuple(pltpu.VMEM((_U * _LANES,), jnp.float32) for _ in range(2)),
            pltpu.SemaphoreType.DMA,
            pltpu.SemaphoreType.DMA,
        ],
    )
    return run(entity_embedding, trig, heads, tails, relations)


_GRP = 128 // _LANES


def _finish_kernel(p_ref, o_ref):
    x = p_ref[...]
    k = lax.broadcasted_iota(jnp.int32, (128, _GRP), 0) // _LANES
    g = lax.broadcasted_iota(jnp.int32, (128, _GRP), 1)
    m = (k == g).astype(jnp.bfloat16)
    s = jax.lax.dot_general(x.astype(jnp.bfloat16), m, (((1,), (0,)), ((), ())),
                            preferred_element_type=jnp.float32)
    o_ref[...] = _MARGIN - jnp.sqrt(s)


def _finish(part):
    p2 = part.reshape(_BATCH // _GRP, 128)
    out = pl.pallas_call(
        _finish_kernel,
        out_shape=jax.ShapeDtypeStruct((_BATCH // _GRP, _GRP), jnp.float32),
    )(p2)
    return out.reshape(_BATCH)


@jax.jit
def kernel(heads, relations, tails, entity_embedding, relation_embedding):
    heads = heads.astype(jnp.int32)
    tails = tails.astype(jnp.int32)
    relations = relations.astype(jnp.int32)
    trig = _trig_table(relation_embedding)
    part = _sc_score(entity_embedding, trig, heads, tails, relations)
    return _finish(part)
